# baseline (speedup 1.0000x reference)
import jax
import jax.numpy as jnp
from jax.experimental import pallas as pl
from jax.experimental.pallas import tpu as pltpu

_ROWS = 16 * 2048
_C = 256
_BLOCK_ROWS = 1024


def _kl_block_kernel(pmu_ref, pls_ref, fmu_ref, fls_ref, out_ref, acc_ref):
    i = pl.program_id(0)

    @pl.when(i == 0)
    def _init():
        acc_ref[...] = jnp.zeros_like(acc_ref)

    pls = pls_ref[...]
    fls = fls_ref[...]
    d = fmu_ref[...] - pmu_ref[...]
    var_f = jnp.exp(2.0 * fls)
    inv_2vp = 0.5 * jnp.exp(-2.0 * pls)
    kl = (pls - fls - 0.5) + (var_f + d * d) * inv_2vp
    acc_ref[...] += jnp.sum(kl.reshape(-1, 8, _C), axis=0)

    @pl.when(i == pl.num_programs(0) - 1)
    def _fin():
        out_ref[...] = jnp.sum(acc_ref[...])[None, None]


def kernel(present_mu, present_log_sigma, future_mu, future_log_sigma):
    pmu = present_mu.reshape(_ROWS, _C)
    pls = present_log_sigma.reshape(_ROWS, _C)
    fmu = future_mu.reshape(_ROWS, _C)
    fls = future_log_sigma.reshape(_ROWS, _C)

    grid = (_ROWS // _BLOCK_ROWS,)
    in_spec = pl.BlockSpec((_BLOCK_ROWS, _C), lambda i: (i, 0))
    out = pl.pallas_call(
        _kl_block_kernel,
        grid=grid,
        in_specs=[in_spec, in_spec, in_spec, in_spec],
        out_specs=pl.BlockSpec((1, 1), lambda i: (0, 0)),
        out_shape=jax.ShapeDtypeStruct((1, 1), jnp.float32),
        scratch_shapes=[pltpu.VMEM((8, _C), jnp.float32)],
    )(pmu, pls, fmu, fls)
    return out[0, 0] / jnp.float32(_ROWS)


# speedup vs baseline: 1.1550x; 1.1550x over previous
import jax
import jax.numpy as jnp
from jax.experimental import pallas as pl
from jax.experimental.pallas import tpu as pltpu

_ROWS = 16 * 2048
_C = 256
_BLOCK_ROWS = 2048


def _kl_block_kernel(pmu_ref, pls_ref, fmu_ref, fls_ref, out_ref, acc_ref):
    i = pl.program_id(0)

    @pl.when(i == 0)
    def _init():
        acc_ref[...] = jnp.zeros_like(acc_ref)

    pls = pls_ref[...]
    fls = fls_ref[...]
    d = fmu_ref[...] - pmu_ref[...]
    var_f = jnp.exp(2.0 * fls)
    inv_2vp = 0.5 * jnp.exp(-2.0 * pls)
    kl = (pls - fls - 0.5) + (var_f + d * d) * inv_2vp
    acc_ref[...] += jnp.sum(kl.reshape(-1, 8, _C), axis=0)

    @pl.when(i == pl.num_programs(0) - 1)
    def _fin():
        out_ref[...] = jnp.sum(acc_ref[...])[None, None]


def kernel(present_mu, present_log_sigma, future_mu, future_log_sigma):
    pmu = present_mu.reshape(_ROWS, _C)
    pls = present_log_sigma.reshape(_ROWS, _C)
    fmu = future_mu.reshape(_ROWS, _C)
    fls = future_log_sigma.reshape(_ROWS, _C)

    grid = (_ROWS // _BLOCK_ROWS,)
    in_spec = pl.BlockSpec((_BLOCK_ROWS, _C), lambda i: (i, 0))
    out = pl.pallas_call(
        _kl_block_kernel,
        grid=grid,
        in_specs=[in_spec, in_spec, in_spec, in_spec],
        out_specs=pl.BlockSpec((1, 1), lambda i: (0, 0)),
        out_shape=jax.ShapeDtypeStruct((1, 1), jnp.float32),
        scratch_shapes=[pltpu.VMEM((8, _C), jnp.float32)],
    )(pmu, pls, fmu, fls)
    return out[0, 0] / jnp.float32(_ROWS)
